# initial kernel scaffold (unmeasured)
import jax
import jax.numpy as jnp
from jax import lax
from jax.experimental import pallas as pl
from jax.experimental.pallas import tpu as pltpu

B, H, D, BS = 32, 16, 128, 32
NB_LOCAL = 256
PAGES_PER_CHUNK = 32
TOK = PAGES_PER_CHUNK * BS
N_CHUNKS = NB_LOCAL // PAGES_PER_CHUNK
NT = 256
NEG = jnp.float32(-1e30)
SCALE = D ** -0.5


def kernel(Q, K, V, bt, lens):
    K2 = K.reshape(NB_LOCAL * BS, H, D)
    V2 = V.reshape(NB_LOCAL * BS, H, D)
    lens2 = lens.reshape(B, 1)

    def body(q_ref, k_ref, v_ref, bt_ref, lens_ref, out_ref,
             acc_ref, stats_ref, acc_rx_ref, stats_rx_ref,
             send_sems, recv_sems):
        c = pl.program_id(0)
        my_x = lax.axis_index("x")
        my_y = lax.axis_index("y")
        peer = (my_x, 1 - my_y)

        @pl.when(c == 0)
        def _():
            bsem = pltpu.get_barrier_semaphore()
            pl.semaphore_signal(bsem, inc=1, device_id=peer,
                                device_id_type=pl.DeviceIdType.MESH)
            pl.semaphore_wait(bsem, 1)
            stats_ref[0] = jnp.full((B, H), NEG, jnp.float32)
            stats_ref[1] = jnp.zeros((B, H), jnp.float32)
            acc_ref[...] = jnp.zeros((H, B, D), jnp.float32)

        base = my_y * NB_LOCAL + c * PAGES_PER_CHUNK
        valid = lax.broadcasted_iota(jnp.int32, (B, NT), 1) < lens_ref[...]
        btm = jnp.where(valid, bt_ref[...], -1)
        pid3 = base + lax.broadcasted_iota(
            jnp.int32, (PAGES_PER_CHUNK, B, NT), 0)
        eq = (btm.reshape(1, B, NT) == pid3)
        cnt_pb = jnp.sum(jnp.where(eq, 1.0, 0.0), axis=2)

        rows = lax.broadcasted_iota(jnp.int32, (PAGES_PER_CHUNK, TOK), 0)
        cols = lax.broadcasted_iota(jnp.int32, (PAGES_PER_CHUNK, TOK), 1)
        E = jnp.where(rows == cols // BS, 1.0, 0.0)
        cnt_k = lax.dot_general(cnt_pb, E, (((0,), (0,)), ((), ())),
                                preferred_element_type=jnp.float32)
        ref_mask = cnt_k > 0.0

        for h in range(H):
            qh = q_ref[:, 0, h, :] * SCALE
            kh = k_ref[:, h, :]
            s = lax.dot_general(qh, kh, (((1,), (1,)), ((), ())),
                                preferred_element_type=jnp.float32)
            s = jnp.where(ref_mask, s, NEG)
            m_old = stats_ref[0, :, h:h + 1]
            l_old = stats_ref[1, :, h:h + 1]
            m_new = jnp.maximum(m_old, jnp.max(s, axis=1, keepdims=True))
            alpha = jnp.exp(m_old - m_new)
            p = jnp.exp(s - m_new) * cnt_k
            l_new = alpha * l_old + jnp.sum(p, axis=1, keepdims=True)
            vh = v_ref[:, h, :]
            pv = jnp.dot(p, vh, preferred_element_type=jnp.float32)
            acc_ref[h] = acc_ref[h] * alpha + pv
            stats_ref[0, :, h:h + 1] = m_new
            stats_ref[1, :, h:h + 1] = l_new

        @pl.when(c == N_CHUNKS - 1)
        def _():
            rdma_acc = pltpu.make_async_remote_copy(
                src_ref=acc_ref, dst_ref=acc_rx_ref,
                send_sem=send_sems.at[0], recv_sem=recv_sems.at[0],
                device_id=peer, device_id_type=pl.DeviceIdType.MESH)
            rdma_st = pltpu.make_async_remote_copy(
                src_ref=stats_ref, dst_ref=stats_rx_ref,
                send_sem=send_sems.at[1], recv_sem=recv_sems.at[1],
                device_id=peer, device_id_type=pl.DeviceIdType.MESH)
            rdma_acc.start()
            rdma_st.start()
            rdma_acc.wait()
            rdma_st.wait()

            for h in range(H):
                m_a = stats_ref[0, :, h:h + 1]
                l_a = stats_ref[1, :, h:h + 1]
                m_b = stats_rx_ref[0, :, h:h + 1]
                l_b = stats_rx_ref[1, :, h:h + 1]
                m_g = jnp.maximum(m_a, m_b)
                wa = jnp.exp(m_a - m_g)
                wb = jnp.exp(m_b - m_g)
                l_g = l_a * wa + l_b * wb
                out_ref[:, 0, h, :] = (
                    acc_ref[h] * wa + acc_rx_ref[h] * wb) / l_g

    return pl.pallas_call(
        body,
        grid=(N_CHUNKS,),
        in_specs=[
            pl.BlockSpec((B, 1, H, D), lambda c: (0, 0, 0, 0)),
            pl.BlockSpec((TOK, H, D), lambda c: (c, 0, 0)),
            pl.BlockSpec((TOK, H, D), lambda c: (c, 0, 0)),
            pl.BlockSpec((B, NT), lambda c: (0, 0)),
            pl.BlockSpec((B, 1), lambda c: (0, 0)),
        ],
        out_specs=pl.BlockSpec((B, 1, H, D), lambda c: (0, 0, 0, 0)),
        out_shape=jax.ShapeDtypeStruct((B, 1, H, D), jnp.float32),
        scratch_shapes=[
            pltpu.VMEM((H, B, D), jnp.float32),
            pltpu.VMEM((2, B, H), jnp.float32),
            pltpu.VMEM((H, B, D), jnp.float32),
            pltpu.VMEM((2, B, H), jnp.float32),
            pltpu.SemaphoreType.DMA((2,)),
            pltpu.SemaphoreType.DMA((2,)),
        ],
        compiler_params=pltpu.CompilerParams(
            dimension_semantics=("arbitrary",),
            collective_id=0,
        ),
    )(Q, K2, V2, bt, lens2)


# baseline (device time: 136902 ns/iter reference)
import jax
import jax.numpy as jnp
from jax import lax
from jax.experimental import pallas as pl
from jax.experimental.pallas import tpu as pltpu

B, H, D, BS = 32, 16, 128, 32
NB_LOCAL = 256
PAGES_PER_CHUNK = 16
TOK = PAGES_PER_CHUNK * BS
N_CHUNKS = NB_LOCAL // PAGES_PER_CHUNK
NT = 256
NEG = -1e30
SCALE = D ** -0.5


def kernel(Q, K, V, bt, lens):
    K2 = K.reshape(NB_LOCAL * BS, H, D)
    V2 = V.reshape(NB_LOCAL * BS, H, D)
    lens2 = lens.reshape(B, 1)

    def body(q_ref, k_ref, v_ref, bt_ref, lens_ref, out_ref,
             acc_ref, stats_ref, acc_rx_ref, stats_rx_ref,
             send_sems, recv_sems):
        c = pl.program_id(0)
        my_x = lax.axis_index("x")
        my_y = lax.axis_index("y")
        peer = (my_x, 1 - my_y)

        @pl.when(c == 0)
        def _():
            bsem = pltpu.get_barrier_semaphore()
            pl.semaphore_signal(bsem, inc=1, device_id=peer,
                                device_id_type=pl.DeviceIdType.MESH)
            pl.semaphore_wait(bsem, 1)
            stats_ref[0] = jnp.full((B, H), NEG, jnp.float32)
            stats_ref[1] = jnp.zeros((B, H), jnp.float32)
            acc_ref[...] = jnp.zeros((H, B, D), jnp.float32)

        base = my_y * NB_LOCAL + c * PAGES_PER_CHUNK
        valid = lax.broadcasted_iota(jnp.int32, (B, NT), 1) < lens_ref[...]
        btm = jnp.where(valid, bt_ref[...], -1)
        pid3 = base + lax.broadcasted_iota(
            jnp.int32, (PAGES_PER_CHUNK, B, NT), 0)
        eq = (btm.reshape(1, B, NT) == pid3)
        cnt_pb = jnp.sum(jnp.where(eq, 1.0, 0.0), axis=2)

        rows = lax.broadcasted_iota(jnp.int32, (PAGES_PER_CHUNK, TOK), 0)
        cols = lax.broadcasted_iota(jnp.int32, (PAGES_PER_CHUNK, TOK), 1)
        E = jnp.where(rows == cols // BS, 1.0, 0.0)
        cnt_k = lax.dot_general(cnt_pb, E, (((0,), (0,)), ((), ())),
                                preferred_element_type=jnp.float32)
        ref_mask = cnt_k > 0.0

        for h in range(H):
            qh = q_ref[:, 0, h, :] * SCALE
            kh = k_ref[:, h, :]
            s = lax.dot_general(qh, kh, (((1,), (1,)), ((), ())),
                                preferred_element_type=jnp.float32)
            s = jnp.where(ref_mask, s, NEG)
            m_old = stats_ref[0, :, h:h + 1]
            l_old = stats_ref[1, :, h:h + 1]
            m_new = jnp.maximum(m_old, jnp.max(s, axis=1, keepdims=True))
            alpha = jnp.exp(m_old - m_new)
            p = jnp.exp(s - m_new) * cnt_k
            l_new = alpha * l_old + jnp.sum(p, axis=1, keepdims=True)
            vh = v_ref[:, h, :]
            pv = jnp.dot(p, vh, preferred_element_type=jnp.float32)
            acc_ref[h] = acc_ref[h] * alpha + pv
            stats_ref[0, :, h:h + 1] = m_new
            stats_ref[1, :, h:h + 1] = l_new

        @pl.when(c == N_CHUNKS - 1)
        def _():
            rdma_acc = pltpu.make_async_remote_copy(
                src_ref=acc_ref, dst_ref=acc_rx_ref,
                send_sem=send_sems.at[0], recv_sem=recv_sems.at[0],
                device_id=peer, device_id_type=pl.DeviceIdType.MESH)
            rdma_st = pltpu.make_async_remote_copy(
                src_ref=stats_ref, dst_ref=stats_rx_ref,
                send_sem=send_sems.at[1], recv_sem=recv_sems.at[1],
                device_id=peer, device_id_type=pl.DeviceIdType.MESH)
            rdma_acc.start()
            rdma_st.start()
            rdma_acc.wait()
            rdma_st.wait()

            for h in range(H):
                m_a = stats_ref[0, :, h:h + 1]
                l_a = stats_ref[1, :, h:h + 1]
                m_b = stats_rx_ref[0, :, h:h + 1]
                l_b = stats_rx_ref[1, :, h:h + 1]
                m_g = jnp.maximum(m_a, m_b)
                wa = jnp.exp(m_a - m_g)
                wb = jnp.exp(m_b - m_g)
                l_g = l_a * wa + l_b * wb
                out_ref[:, 0, h, :] = (
                    acc_ref[h] * wa + acc_rx_ref[h] * wb) / l_g

    return pl.pallas_call(
        body,
        grid=(N_CHUNKS,),
        in_specs=[
            pl.BlockSpec((B, 1, H, D), lambda c: (0, 0, 0, 0)),
            pl.BlockSpec((TOK, H, D), lambda c: (c, 0, 0)),
            pl.BlockSpec((TOK, H, D), lambda c: (c, 0, 0)),
            pl.BlockSpec((B, NT), lambda c: (0, 0)),
            pl.BlockSpec((B, 1), lambda c: (0, 0)),
        ],
        out_specs=pl.BlockSpec((B, 1, H, D), lambda c: (0, 0, 0, 0)),
        out_shape=jax.ShapeDtypeStruct((B, 1, H, D), jnp.float32),
        scratch_shapes=[
            pltpu.VMEM((H, B, D), jnp.float32),
            pltpu.VMEM((2, B, H), jnp.float32),
            pltpu.VMEM((H, B, D), jnp.float32),
            pltpu.VMEM((2, B, H), jnp.float32),
            pltpu.SemaphoreType.DMA((2,)),
            pltpu.SemaphoreType.DMA((2,)),
        ],
        compiler_params=pltpu.CompilerParams(
            dimension_semantics=("arbitrary",),
            collective_id=0,
        ),
    )(Q, K2, V2, bt, lens2)
